# baseline (speedup 1.0000x reference)
import functools

import jax
import jax.numpy as jnp
from jax import lax
from jax.experimental import pallas as pl
from jax.experimental.pallas import tpu as pltpu
from jax.experimental.pallas import tpu_sc as plsc

B = 4
T = 2048
D = 128
NP = 8
NW = 32
TCHUNK = T // NW
LANES = 16


def _emb_body(idx_hbm, wte_hbm, wpe_hbm, prompts_hbm, out_hbm,
              idx_v, rows_v, wpe_v, prompts_v, isem, gsem, ssem, wsem):
    cid = lax.axis_index("c")
    sid = lax.axis_index("s")
    wid = sid * 2 + cid
    t0 = wid * TCHUNK

    icopies = [
        pltpu.async_copy(idx_hbm.at[b, pl.ds(t0, TCHUNK)], idx_v.at[b], isem)
        for b in range(B)
    ]
    wcopy = pltpu.async_copy(wpe_hbm.at[pl.ds(t0, TCHUNK)], wpe_v, wsem)
    is_prompt_worker = wid < B

    @pl.when(is_prompt_worker)
    def _():
        pltpu.async_copy(prompts_hbm, prompts_v, isem).wait()

    HALF = TCHUNK // 2
    gcopies = []
    for b in range(B):
        icopies[b].wait()
        for h in range(2):
            gcopies.append(pltpu.async_copy(
                wte_hbm.at[idx_v.at[b, pl.ds(h * HALF, HALF)]],
                rows_v.at[b, pl.ds(h * HALF, HALF)], gsem))

    @pl.when(is_prompt_worker)
    def _():
        pltpu.async_copy(prompts_v, out_hbm.at[wid, pl.ds(0, NP)],
                         ssem).wait()

    wcopy.wait()

    scopies = []
    for b in range(B):
        for h in range(2):
            gcopies[b * 2 + h].wait()

            def add_row(i, _, b=b, h=h):
                row = h * HALF + i
                for j in range(D // LANES):
                    s = pl.ds(j * LANES, LANES)
                    plsc.addupdate(rows_v.at[b, row, s], wpe_v[row, s])
                return _

            lax.fori_loop(0, HALF, add_row, None)
            scopies.append(pltpu.async_copy(
                rows_v.at[b, pl.ds(h * HALF, HALF)],
                out_hbm.at[b, pl.ds(NP + t0 + h * HALF, HALF)], ssem))
    for c in scopies:
        c.wait()


@jax.jit
def kernel(idx, wte, wpe, reasoning_prompts):
    mesh = plsc.VectorSubcoreMesh(core_axis_name="c", subcore_axis_name="s")
    run = functools.partial(
        pl.kernel,
        out_type=jax.ShapeDtypeStruct((B, NP + T, D), jnp.float32),
        mesh=mesh,
        scratch_types=[
            pltpu.VMEM((B, TCHUNK), jnp.int32),
            pltpu.VMEM((B, TCHUNK, D), jnp.float32),
            pltpu.VMEM((TCHUNK, D), jnp.float32),
            pltpu.VMEM((NP, D), jnp.float32),
            pltpu.SemaphoreType.DMA,
            pltpu.SemaphoreType.DMA,
            pltpu.SemaphoreType.DMA,
            pltpu.SemaphoreType.DMA,
        ],
    )(_emb_body)
    return run(idx.astype(jnp.int32), wte, wpe, reasoning_prompts)


# speedup vs baseline: 1.0080x; 1.0080x over previous
import functools

import jax
import jax.numpy as jnp
from jax import lax
from jax.experimental import pallas as pl
from jax.experimental.pallas import tpu as pltpu
from jax.experimental.pallas import tpu_sc as plsc

B = 4
T = 2048
D = 128
NP = 8
NW = 32
TCHUNK = T // NW
LANES = 16


def _emb_body(idx_hbm, wte_hbm, wpe_hbm, prompts_hbm, out_hbm,
              idx_v, rows_v, wpe_v, prompts_v, isem, gsem, ssem, wsem):
    cid = lax.axis_index("c")
    sid = lax.axis_index("s")
    wid = sid * 2 + cid
    t0 = wid * TCHUNK

    icopies = [
        pltpu.async_copy(idx_hbm.at[b, pl.ds(t0, TCHUNK)], idx_v.at[b], isem)
        for b in range(B)
    ]
    wcopy = pltpu.async_copy(wpe_hbm.at[pl.ds(t0, TCHUNK)], wpe_v, wsem)
    is_prompt_worker = wid < B

    @pl.when(is_prompt_worker)
    def _():
        pltpu.async_copy(prompts_hbm, prompts_v, isem).wait()

    gcopies = []
    for b in range(B):
        icopies[b].wait()
        gcopies.append(
            pltpu.async_copy(wte_hbm.at[idx_v.at[b]], rows_v.at[b], gsem))

    @pl.when(is_prompt_worker)
    def _():
        pltpu.async_copy(prompts_v, out_hbm.at[wid, pl.ds(0, NP)],
                         ssem).wait()

    wcopy.wait()

    scopies = []
    for b in range(B):
        gcopies[b].wait()

        def add_row(i, _, b=b):
            for j in range(D // LANES):
                s = pl.ds(j * LANES, LANES)
                plsc.addupdate(rows_v.at[b, i, s], wpe_v[i, s])
            return _

        lax.fori_loop(0, TCHUNK, add_row, None)
        scopies.append(
            pltpu.async_copy(rows_v.at[b],
                             out_hbm.at[b, pl.ds(NP + t0, TCHUNK)], ssem))
    for c in scopies:
        c.wait()


@jax.jit
def kernel(idx, wte, wpe, reasoning_prompts):
    mesh = plsc.VectorSubcoreMesh(core_axis_name="c", subcore_axis_name="s")
    run = functools.partial(
        pl.kernel,
        out_type=jax.ShapeDtypeStruct((B, NP + T, D), jnp.float32),
        mesh=mesh,
        scratch_types=[
            pltpu.VMEM((B, TCHUNK), jnp.int32),
            pltpu.VMEM((B, TCHUNK, D), jnp.float32),
            pltpu.VMEM((TCHUNK, D), jnp.float32),
            pltpu.VMEM((NP, D), jnp.float32),
            pltpu.SemaphoreType.DMA,
            pltpu.SemaphoreType.DMA,
            pltpu.SemaphoreType.DMA,
            pltpu.SemaphoreType.DMA,
        ],
    )(_emb_body)
    return run(idx.astype(jnp.int32), wte, wpe, reasoning_prompts)
